# baseline (speedup 1.0000x reference)
import functools

import jax
import jax.numpy as jnp
from jax import lax
from jax.experimental import pallas as pl
from jax.experimental.pallas import tpu as pltpu
from jax.experimental.pallas import tpu_sc as plsc

N = 4096
T_ = 32
A_ = 16
TA = 512
D = 256
H = 2048
K = 8192
NB = 512
KB = 2048
GRID = N // NB


CH = H // (K // KB)


def _enc_vq0_body(x_ref, w1_ref, b1_ref, w2_ref, b2_ref, w3_ref, b3_ref,
                  cb_ref, rep_ref, idx_ref, scr_ref):
    n = pl.program_id(0)
    x = x_ref[...].reshape(NB, TA)
    h1 = jnp.maximum(
        jnp.dot(x, w1_ref[...], preferred_element_type=jnp.float32) + b1_ref[...], 0.0)
    prev = scr_ref[pl.ds((n + 1) % 2, 1)].reshape(NB, D)
    r2 = -2.0 * prev
    rep_ref[...] = jnp.broadcast_to(b3_ref[...], (NB, D))

    def step(j, carry):
        bmin, barg = carry
        w2j = w2_ref[:, pl.ds(j * CH, CH)]
        b2j = b2_ref[:, pl.ds(j * CH, CH)]
        h2j = jnp.maximum(
            jnp.dot(h1, w2j, preferred_element_type=jnp.float32) + b2j, 0.0)
        w3j = w3_ref[pl.ds(j * CH, CH), :]
        rep_ref[...] += jnp.dot(h2j, w3j, preferred_element_type=jnp.float32)
        cbk = cb_ref[0, pl.ds(j * KB, KB), :]
        s = lax.dot_general(cbk, r2, (((1,), (1,)), ((), ())),
                            preferred_element_type=jnp.float32)
        cn = jnp.sum(cbk * cbk, axis=1, keepdims=True)
        s = s + cn
        lmin = jnp.min(s, axis=0, keepdims=True)
        ii = lax.broadcasted_iota(jnp.int32, (KB, NB), 0).astype(jnp.float32)
        cand = jnp.where(s == lmin, ii, jnp.float32(KB))
        larg = jnp.min(cand, axis=0, keepdims=True).astype(jnp.int32) + j * KB
        imp = lmin < bmin
        return jnp.where(imp, lmin, bmin), jnp.where(imp, larg, barg)

    init = (jnp.full((1, NB), jnp.inf, jnp.float32),
            jnp.zeros((1, NB), jnp.int32))
    _, barg = lax.fori_loop(0, K // KB, step, init)
    idx_ref[0] = barg
    scr_ref[pl.ds(n % 2, 1)] = rep_ref[...].reshape(1, NB, D)


def _encode_vq0(state, w1, b1, w2, b2, w3, b3, codebooks):
    full = lambda shape: pl.BlockSpec(shape, lambda n: tuple(0 for _ in shape))
    last = GRID - 1
    rep, idx3 = pl.pallas_call(
        _enc_vq0_body,
        grid=(GRID + 1,),
        in_specs=[
            pl.BlockSpec((NB, T_, A_), lambda n: (jnp.minimum(n, last), 0, 0)),
            full((TA, H)), full((1, H)),
            full((H, H)), full((1, H)),
            full((H, D)), full((1, D)),
            full((1, K, D)),
        ],
        out_specs=[
            pl.BlockSpec((NB, D), lambda n: (jnp.minimum(n, last), 0)),
            pl.BlockSpec((1, 1, NB), lambda n: (jnp.maximum(n - 1, 0), 0, 0)),
        ],
        out_shape=[
            jax.ShapeDtypeStruct((N, D), jnp.float32),
            jax.ShapeDtypeStruct((GRID, 1, NB), jnp.int32),
        ],
        scratch_shapes=[pltpu.VMEM((2, NB, D), jnp.float32)],
    )(state, w1, b1, w2, b2, w3, b3, codebooks)
    return rep, idx3.reshape(N)


def _argmin_over_k(res, cb_ref, idx_ref):
    nsteps = K // KB
    r2 = -2.0 * res
    ii = lax.broadcasted_iota(jnp.int32, (KB, NB), 0).astype(jnp.float32)

    def step(k, carry):
        bmin, barg = carry
        cbk = cb_ref[0, pl.ds(k * KB, KB), :]
        s = lax.dot_general(cbk, r2, (((1,), (1,)), ((), ())),
                            preferred_element_type=jnp.float32)
        cn = jnp.sum(cbk * cbk, axis=1, keepdims=True)
        s = s + cn
        lmin = jnp.min(s, axis=0, keepdims=True)
        cand = jnp.where(s == lmin, ii, jnp.float32(KB))
        larg_f = jnp.min(cand, axis=0, keepdims=True)
        larg = larg_f.astype(jnp.int32) + k * KB
        improved = lmin < bmin
        return (jnp.where(improved, lmin, bmin),
                jnp.where(improved, larg, barg))

    init = (jnp.full((1, NB), jnp.inf, jnp.float32),
            jnp.zeros((1, NB), jnp.int32))
    _, barg = lax.fori_loop(0, nsteps, step, init)
    idx_ref[0] = barg


def _vq1_body(rep_ref, q0_ref, cb_ref, idx_ref):
    _argmin_over_k(rep_ref[...] - q0_ref[...], cb_ref, idx_ref)


def _dec_body(x_ref, rep_ref, q0_ref, q1_ref,
              w1_ref, b1_ref, w2_ref, b2_ref, w3_ref, b3_ref, loss_ref):
    q0 = q0_ref[...]
    q1 = q1_ref[...]
    rep = rep_ref[...]
    sv = q0 + q1
    h = jnp.maximum(
        jnp.dot(sv, w1_ref[...], preferred_element_type=jnp.float32) + b1_ref[...], 0.0)
    h = jnp.maximum(
        jnp.dot(h, w2_ref[...], preferred_element_type=jnp.float32) + b2_ref[...], 0.0)
    out = jnp.dot(h, w3_ref[...], preferred_element_type=jnp.float32) + b3_ref[...]

    def _sum2d(a):
        return jnp.sum(jnp.sum(a, axis=1, keepdims=True), axis=0, keepdims=True)

    sabs = _sum2d(jnp.abs(x_ref[...] - out))
    e0 = q0 - rep
    e1 = q1 - (rep - q0)
    sse = _sum2d(e0 * e0) + _sum2d(e1 * e1)
    contrib = sabs / (N * TA) + 5.0 * sse / (N * D)

    @pl.when(pl.program_id(0) == 0)
    def _init():
        loss_ref[...] = jnp.zeros_like(loss_ref)

    loss_ref[...] += contrib


MB = 1024


def _encode(x, w1, b1, w2, b2, w3, b3):
    full = lambda shape: pl.BlockSpec(shape, lambda n: (0, 0))
    return pl.pallas_call(
        _enc_body,
        grid=(N // MB,),
        in_specs=[
            pl.BlockSpec((MB, TA), lambda n: (n, 0)),
            full((TA, H)), full((1, H)),
            full((H, H)), full((1, H)),
            full((H, D)), full((1, D)),
        ],
        out_specs=pl.BlockSpec((MB, D), lambda n: (n, 0)),
        out_shape=jax.ShapeDtypeStruct((N, D), jnp.float32),
    )(x, w1, b1, w2, b2, w3, b3)


def _vq_argmin(rep, codebooks, q, q0):
    cb_spec = pl.BlockSpec((1, K, D), lambda n: (q, 0, 0))
    row_spec = pl.BlockSpec((NB, D), lambda n: (n, 0))
    idx_spec = pl.BlockSpec((1, 1, NB), lambda n: (n, 0, 0))
    out_shape = jax.ShapeDtypeStruct((GRID, 1, NB), jnp.int32)
    idx3 = pl.pallas_call(
        _vq1_body, grid=(GRID,),
        in_specs=[row_spec, row_spec, cb_spec],
        out_specs=idx_spec, out_shape=out_shape,
    )(rep, q0, codebooks)
    return idx3.reshape(N)


def _sc_gather(table, idx):
    info = plsc.get_sparse_core_info()
    nw = info.num_cores * info.num_subcores
    b_per_w = N // nw
    mesh = plsc.VectorSubcoreMesh(core_axis_name="c", subcore_axis_name="s")

    @functools.partial(
        pl.kernel, mesh=mesh,
        out_type=jax.ShapeDtypeStruct((N, D), jnp.float32),
        scratch_types=[
            pltpu.VMEM((b_per_w,), jnp.int32),
            pltpu.VMEM((b_per_w, D), jnp.float32),
            pltpu.SemaphoreType.DMA,
        ],
    )
    def gather_k(table_hbm, idx_hbm, out_hbm, idx_v, rows_v, sem):
        wid = lax.axis_index("s") * info.num_cores + lax.axis_index("c")
        base = wid * b_per_w
        pltpu.sync_copy(idx_hbm.at[pl.ds(base, b_per_w)], idx_v)
        pltpu.async_copy(table_hbm.at[idx_v], rows_v, sem).wait()
        pltpu.sync_copy(rows_v, out_hbm.at[pl.ds(base, b_per_w)])

    return gather_k(table, idx)


def _decode_loss(x, rep, q0, q1, w1, b1, w2, b2, w3, b3):
    full = lambda shape: pl.BlockSpec(shape, lambda n: (0, 0))
    row = lambda w: pl.BlockSpec((MB, w), lambda n: (n, 0))
    loss = pl.pallas_call(
        _dec_body,
        grid=(N // MB,),
        in_specs=[
            row(TA), row(D), row(D), row(D),
            full((D, H)), full((1, H)),
            full((H, H)), full((1, H)),
            full((H, TA)), full((1, TA)),
        ],
        out_specs=pl.BlockSpec((1, 1), lambda n: (0, 0)),
        out_shape=jax.ShapeDtypeStruct((1, 1), jnp.float32),
    )(x, rep, q0, q1, w1, b1, w2, b2, w3, b3)
    return loss[0, 0]


def kernel(state, enc_w1, enc_b1, enc_w2, enc_b2, enc_w3, enc_b3,
           dec_w1, dec_b1, dec_w2, dec_b2, dec_w3, dec_b3, codebooks):
    x = state.reshape(N, TA)
    cb2d = codebooks.reshape(2 * K, D)
    rep, idx0 = _encode_vq0(state, enc_w1, enc_b1.reshape(1, H),
                            enc_w2, enc_b2.reshape(1, H),
                            enc_w3, enc_b3.reshape(1, D), codebooks)
    quant0 = _sc_gather(cb2d, idx0)
    idx1 = _vq_argmin(rep, codebooks, 1, quant0)
    quant1 = _sc_gather(cb2d, idx1 + K)
    rep_loss = _decode_loss(x, rep, quant0, quant1,
                            dec_w1, dec_b1.reshape(1, H),
                            dec_w2, dec_b2.reshape(1, H),
                            dec_w3, dec_b3.reshape(1, TA))
    vq_code = jnp.stack([idx0, idx1], axis=-1)
    return rep_loss, vq_code


# speedup vs baseline: 1.2483x; 1.2483x over previous
import functools

import jax
import jax.numpy as jnp
from jax import lax
from jax.experimental import pallas as pl
from jax.experimental.pallas import tpu as pltpu
from jax.experimental.pallas import tpu_sc as plsc

N = 4096
T_ = 32
A_ = 16
TA = 512
D = 256
H = 2048
K = 8192
NB = 512
KB = 2048
GRID = N // NB
MB = 1024


def _enc_body(x_ref, w1_ref, b1_ref, w2_ref, b2_ref, w3_ref, b3_ref, rep_ref):
    x = x_ref[...]
    h = jnp.maximum(
        jnp.dot(x, w1_ref[...], preferred_element_type=jnp.float32) + b1_ref[...], 0.0)
    h = jnp.maximum(
        jnp.dot(h, w2_ref[...], preferred_element_type=jnp.float32) + b2_ref[...], 0.0)
    rep_ref[...] = (
        jnp.dot(h, w3_ref[...], preferred_element_type=jnp.float32) + b3_ref[...])


def _argmin_over_k(res, cb_ref, idx_ref):
    nsteps = K // KB
    r2 = -2.0 * res
    ii = lax.broadcasted_iota(jnp.int32, (KB, NB), 0).astype(jnp.float32)

    def step(k, carry):
        bmin, barg = carry
        cbk = cb_ref[0, pl.ds(k * KB, KB), :]
        s = lax.dot_general(cbk, r2, (((1,), (1,)), ((), ())),
                            preferred_element_type=jnp.float32)
        cn = jnp.sum(cbk * cbk, axis=1, keepdims=True)
        s = s + cn
        lmin = jnp.min(s, axis=0, keepdims=True)
        cand = jnp.where(s == lmin, ii, jnp.float32(KB))
        larg_f = jnp.min(cand, axis=0, keepdims=True)
        larg = larg_f.astype(jnp.int32) + k * KB
        improved = lmin < bmin
        return (jnp.where(improved, lmin, bmin),
                jnp.where(improved, larg, barg))

    init = (jnp.full((1, NB), jnp.inf, jnp.float32),
            jnp.zeros((1, NB), jnp.int32))
    _, barg = lax.fori_loop(0, nsteps, step, init)
    idx_ref[0] = barg


def _vq0_body(rep_ref, cb_ref, idx_ref):
    _argmin_over_k(rep_ref[...], cb_ref, idx_ref)


def _vq1_body(rep_ref, q0_ref, cb_ref, idx_ref):
    _argmin_over_k(rep_ref[...] - q0_ref[...], cb_ref, idx_ref)


def _dec_body(x_ref, rep_ref, q0_ref, q1_ref,
              w1_ref, b1_ref, w2_ref, b2_ref, w3_ref, b3_ref, loss_ref):
    q0 = q0_ref[...]
    q1 = q1_ref[...]
    rep = rep_ref[...]
    sv = q0 + q1
    h = jnp.maximum(
        jnp.dot(sv, w1_ref[...], preferred_element_type=jnp.float32) + b1_ref[...], 0.0)
    h = jnp.maximum(
        jnp.dot(h, w2_ref[...], preferred_element_type=jnp.float32) + b2_ref[...], 0.0)
    out = jnp.dot(h, w3_ref[...], preferred_element_type=jnp.float32) + b3_ref[...]

    def _sum2d(a):
        return jnp.sum(jnp.sum(a, axis=1, keepdims=True), axis=0, keepdims=True)

    x = x_ref[...]
    sabs = _sum2d(jnp.abs(x - out))
    e0 = q0 - rep
    e1 = q1 - (rep - q0)
    sse = _sum2d(e0 * e0) + _sum2d(e1 * e1)
    contrib = sabs / (N * TA) + 5.0 * sse / (N * D)

    @pl.when(pl.program_id(0) == 0)
    def _init():
        loss_ref[...] = jnp.zeros_like(loss_ref)

    loss_ref[...] += contrib


def _encode(x, w1, b1, w2, b2, w3, b3):
    full = lambda shape: pl.BlockSpec(shape, lambda n: tuple(0 for _ in shape))
    return pl.pallas_call(
        _enc_body,
        grid=(N // MB,),
        in_specs=[
            pl.BlockSpec((MB, TA), lambda n: (n, 0)),
            full((TA, H)), full((1, H)),
            full((H, H)), full((1, H)),
            full((H, D)), full((1, D)),
        ],
        out_specs=pl.BlockSpec((MB, D), lambda n: (n, 0)),
        out_shape=jax.ShapeDtypeStruct((N, D), jnp.float32),
    )(x, w1, b1, w2, b2, w3, b3)


def _vq_argmin(rep, codebooks, q, q0=None):
    cb_spec = pl.BlockSpec((1, K, D), lambda n: (q, 0, 0))
    row_spec = pl.BlockSpec((NB, D), lambda n: (n, 0))
    idx_spec = pl.BlockSpec((1, 1, NB), lambda n: (n, 0, 0))
    out_shape = jax.ShapeDtypeStruct((GRID, 1, NB), jnp.int32)
    if q0 is None:
        idx3 = pl.pallas_call(
            _vq0_body, grid=(GRID,),
            in_specs=[row_spec, cb_spec],
            out_specs=idx_spec, out_shape=out_shape,
        )(rep, codebooks)
    else:
        idx3 = pl.pallas_call(
            _vq1_body, grid=(GRID,),
            in_specs=[row_spec, row_spec, cb_spec],
            out_specs=idx_spec, out_shape=out_shape,
        )(rep, q0, codebooks)
    return idx3.reshape(N)


def _sc_gather(table, idx):
    info = plsc.get_sparse_core_info()
    nw = info.num_cores * info.num_subcores
    b_per_w = N // nw
    mesh = plsc.VectorSubcoreMesh(core_axis_name="c", subcore_axis_name="s")

    @functools.partial(
        pl.kernel, mesh=mesh,
        out_type=jax.ShapeDtypeStruct((N, D), jnp.float32),
        scratch_types=[
            pltpu.VMEM((b_per_w,), jnp.int32),
            pltpu.VMEM((b_per_w, D), jnp.float32),
            pltpu.SemaphoreType.DMA,
        ],
    )
    def gather_k(table_hbm, idx_hbm, out_hbm, idx_v, rows_v, sem):
        wid = lax.axis_index("s") * info.num_cores + lax.axis_index("c")
        base = wid * b_per_w
        pltpu.sync_copy(idx_hbm.at[pl.ds(base, b_per_w)], idx_v)
        pltpu.async_copy(table_hbm.at[idx_v], rows_v, sem).wait()
        pltpu.sync_copy(rows_v, out_hbm.at[pl.ds(base, b_per_w)])

    return gather_k(table, idx)


def _decode_loss(x, rep, q0, q1, w1, b1, w2, b2, w3, b3):
    full = lambda shape: pl.BlockSpec(shape, lambda n: tuple(0 for _ in shape))
    row = lambda w: pl.BlockSpec((MB, w), lambda n: (n, 0))
    loss = pl.pallas_call(
        _dec_body,
        grid=(N // MB,),
        in_specs=[
            pl.BlockSpec((MB, TA), lambda n: (n, 0)),
            row(D), row(D), row(D),
            full((D, H)), full((1, H)),
            full((H, H)), full((1, H)),
            full((H, TA)), full((1, TA)),
        ],
        out_specs=pl.BlockSpec((1, 1), lambda n: (0, 0)),
        out_shape=jax.ShapeDtypeStruct((1, 1), jnp.float32),
    )(x, rep, q0, q1, w1, b1, w2, b2, w3, b3)
    return loss[0, 0]


def kernel(state, enc_w1, enc_b1, enc_w2, enc_b2, enc_w3, enc_b3,
           dec_w1, dec_b1, dec_w2, dec_b2, dec_w3, dec_b3, codebooks):
    x = state.reshape(N, TA)
    cb2d = codebooks.reshape(2 * K, D)
    rep = _encode(x, enc_w1, enc_b1.reshape(1, H), enc_w2,
                  enc_b2.reshape(1, H), enc_w3, enc_b3.reshape(1, D))
    idx0 = _vq_argmin(rep, codebooks, 0)
    quant0 = _sc_gather(cb2d, idx0)
    idx1 = _vq_argmin(rep, codebooks, 1, q0=quant0)
    quant1 = _sc_gather(cb2d, idx1 + K)
    rep_loss = _decode_loss(x, rep, quant0, quant1,
                            dec_w1, dec_b1.reshape(1, H),
                            dec_w2, dec_b2.reshape(1, H),
                            dec_w3, dec_b3.reshape(1, TA))
    vq_code = jnp.stack([idx0, idx1], axis=-1)
    return rep_loss, vq_code


# speedup vs baseline: 1.2669x; 1.0149x over previous
import functools

import jax
import jax.numpy as jnp
from jax import lax
from jax.experimental import pallas as pl
from jax.experimental.pallas import tpu as pltpu
from jax.experimental.pallas import tpu_sc as plsc

N = 4096
T_ = 32
A_ = 16
TA = 512
D = 256
H = 2048
K = 8192
NB = 512
KB = 2048
GRID = N // NB
MB = 1024


def _enc_body(x_ref, w1_ref, b1_ref, w2_ref, b2_ref, w3_ref, b3_ref, rep_ref):
    x = x_ref[...]
    h = jnp.maximum(
        jnp.dot(x, w1_ref[...], preferred_element_type=jnp.float32) + b1_ref[...], 0.0)
    h = jnp.maximum(
        jnp.dot(h, w2_ref[...], preferred_element_type=jnp.float32) + b2_ref[...], 0.0)
    rep_ref[...] = (
        jnp.dot(h, w3_ref[...], preferred_element_type=jnp.float32) + b3_ref[...])


def _argmin_over_k(res, cb_ref, idx_ref):
    nsteps = K // KB
    r2 = -2.0 * res
    rn = jnp.transpose(jnp.sum(res * res, axis=1, keepdims=True))
    ii = lax.broadcasted_iota(jnp.int32, (KB, NB), 0).astype(jnp.float32)

    def step(k, carry):
        bmin, barg = carry
        cbk = cb_ref[0, pl.ds(k * KB, KB), :]
        m2 = lax.dot_general(cbk, r2, (((1,), (1,)), ((), ())),
                             preferred_element_type=jnp.float32)
        cn = jnp.sum(cbk * cbk, axis=1, keepdims=True)
        s = (rn + m2) + cn
        lmin = jnp.min(s, axis=0, keepdims=True)
        cand = jnp.where(s == lmin, ii, jnp.float32(KB))
        larg_f = jnp.min(cand, axis=0, keepdims=True)
        larg = larg_f.astype(jnp.int32) + k * KB
        improved = lmin < bmin
        return (jnp.where(improved, lmin, bmin),
                jnp.where(improved, larg, barg))

    init = (jnp.full((1, NB), jnp.inf, jnp.float32),
            jnp.zeros((1, NB), jnp.int32))
    _, barg = lax.fori_loop(0, nsteps, step, init)
    idx_ref[0] = barg


def _vq0_body(rep_ref, cb_ref, idx_ref):
    _argmin_over_k(rep_ref[...], cb_ref, idx_ref)


def _vq1_body(rep_ref, q0_ref, cb_ref, idx_ref):
    _argmin_over_k(rep_ref[...] - q0_ref[...], cb_ref, idx_ref)


def _dec_body(x_ref, rep_ref, q0_ref, q1_ref,
              w1_ref, b1_ref, w2_ref, b2_ref, w3_ref, b3_ref, loss_ref):
    q0 = q0_ref[...]
    q1 = q1_ref[...]
    rep = rep_ref[...]
    sv = q0 + q1
    h = jnp.maximum(
        jnp.dot(sv, w1_ref[...], preferred_element_type=jnp.float32) + b1_ref[...], 0.0)
    h = jnp.maximum(
        jnp.dot(h, w2_ref[...], preferred_element_type=jnp.float32) + b2_ref[...], 0.0)
    out = jnp.dot(h, w3_ref[...], preferred_element_type=jnp.float32) + b3_ref[...]

    def _sum2d(a):
        return jnp.sum(jnp.sum(a, axis=1, keepdims=True), axis=0, keepdims=True)

    x = x_ref[...]
    sabs = _sum2d(jnp.abs(x - out))
    e0 = q0 - rep
    e1 = q1 - (rep - q0)
    sse = _sum2d(e0 * e0) + _sum2d(e1 * e1)
    contrib = sabs / (N * TA) + 5.0 * sse / (N * D)

    @pl.when(pl.program_id(0) == 0)
    def _init():
        loss_ref[...] = jnp.zeros_like(loss_ref)

    loss_ref[...] += contrib


def _encode(x, w1, b1, w2, b2, w3, b3):
    full = lambda shape: pl.BlockSpec(shape, lambda n: tuple(0 for _ in shape))
    return pl.pallas_call(
        _enc_body,
        grid=(N // MB,),
        in_specs=[
            pl.BlockSpec((MB, TA), lambda n: (n, 0)),
            full((TA, H)), full((1, H)),
            full((H, H)), full((1, H)),
            full((H, D)), full((1, D)),
        ],
        out_specs=pl.BlockSpec((MB, D), lambda n: (n, 0)),
        out_shape=jax.ShapeDtypeStruct((N, D), jnp.float32),
    )(x, w1, b1, w2, b2, w3, b3)


def _vq_argmin(rep, codebooks, q, q0=None):
    cb_spec = pl.BlockSpec((1, K, D), lambda n: (q, 0, 0))
    row_spec = pl.BlockSpec((NB, D), lambda n: (n, 0))
    idx_spec = pl.BlockSpec((1, 1, NB), lambda n: (n, 0, 0))
    out_shape = jax.ShapeDtypeStruct((GRID, 1, NB), jnp.int32)
    if q0 is None:
        idx3 = pl.pallas_call(
            _vq0_body, grid=(GRID,),
            in_specs=[row_spec, cb_spec],
            out_specs=idx_spec, out_shape=out_shape,
        )(rep, codebooks)
    else:
        idx3 = pl.pallas_call(
            _vq1_body, grid=(GRID,),
            in_specs=[row_spec, row_spec, cb_spec],
            out_specs=idx_spec, out_shape=out_shape,
        )(rep, q0, codebooks)
    return idx3.reshape(N)


def _sc_gather(table, idx):
    info = plsc.get_sparse_core_info()
    nw = info.num_cores * info.num_subcores
    b_per_w = N // nw
    mesh = plsc.VectorSubcoreMesh(core_axis_name="c", subcore_axis_name="s")

    @functools.partial(
        pl.kernel, mesh=mesh,
        out_type=jax.ShapeDtypeStruct((N, D), jnp.float32),
        scratch_types=[
            pltpu.VMEM((b_per_w,), jnp.int32),
            pltpu.VMEM((b_per_w, D), jnp.float32),
            pltpu.SemaphoreType.DMA,
        ],
    )
    def gather_k(table_hbm, idx_hbm, out_hbm, idx_v, rows_v, sem):
        wid = lax.axis_index("s") * info.num_cores + lax.axis_index("c")
        base = wid * b_per_w
        pltpu.sync_copy(idx_hbm.at[pl.ds(base, b_per_w)], idx_v)
        pltpu.async_copy(table_hbm.at[idx_v], rows_v, sem).wait()
        pltpu.sync_copy(rows_v, out_hbm.at[pl.ds(base, b_per_w)])

    return gather_k(table, idx)


def _decode_loss(x, rep, q0, q1, w1, b1, w2, b2, w3, b3):
    full = lambda shape: pl.BlockSpec(shape, lambda n: tuple(0 for _ in shape))
    row = lambda w: pl.BlockSpec((MB, w), lambda n: (n, 0))
    loss = pl.pallas_call(
        _dec_body,
        grid=(N // MB,),
        in_specs=[
            pl.BlockSpec((MB, TA), lambda n: (n, 0)),
            row(D), row(D), row(D),
            full((D, H)), full((1, H)),
            full((H, H)), full((1, H)),
            full((H, TA)), full((1, TA)),
        ],
        out_specs=pl.BlockSpec((1, 1), lambda n: (0, 0)),
        out_shape=jax.ShapeDtypeStruct((1, 1), jnp.float32),
    )(x, rep, q0, q1, w1, b1, w2, b2, w3, b3)
    return loss[0, 0]


def kernel(state, enc_w1, enc_b1, enc_w2, enc_b2, enc_w3, enc_b3,
           dec_w1, dec_b1, dec_w2, dec_b2, dec_w3, dec_b3, codebooks):
    x = state.reshape(N, TA)
    cb2d = codebooks.reshape(2 * K, D)
    rep = _encode(x, enc_w1, enc_b1.reshape(1, H), enc_w2,
                  enc_b2.reshape(1, H), enc_w3, enc_b3.reshape(1, D))
    idx0 = _vq_argmin(rep, codebooks, 0)
    quant0 = _sc_gather(cb2d, idx0)
    idx1 = _vq_argmin(rep, codebooks, 1, q0=quant0)
    quant1 = _sc_gather(cb2d, idx1 + K)
    rep_loss = _decode_loss(x, rep, quant0, quant1,
                            dec_w1, dec_b1.reshape(1, H),
                            dec_w2, dec_b2.reshape(1, H),
                            dec_w3, dec_b3.reshape(1, TA))
    vq_code = jnp.stack([idx0, idx1], axis=-1)
    return rep_loss, vq_code


# speedup vs baseline: 1.3203x; 1.0421x over previous
import functools

import jax
import jax.numpy as jnp
from jax import lax
from jax.experimental import pallas as pl
from jax.experimental.pallas import tpu as pltpu
from jax.experimental.pallas import tpu_sc as plsc

N = 4096
T_ = 32
A_ = 16
TA = 512
D = 256
H = 2048
K = 8192
NB = 512
KB = 2048
GRID = N // NB
MB = 1024


def _enc_body(x_ref, w1_ref, b1_ref, w2_ref, b2_ref, w3_ref, b3_ref, rep_ref):
    x = x_ref[...]
    h = jnp.maximum(
        jnp.dot(x, w1_ref[...], preferred_element_type=jnp.float32) + b1_ref[...], 0.0)
    h = jnp.maximum(
        jnp.dot(h, w2_ref[...], preferred_element_type=jnp.float32) + b2_ref[...], 0.0)
    rep_ref[...] = (
        jnp.dot(h, w3_ref[...], preferred_element_type=jnp.float32) + b3_ref[...])


def _argmin_over_k(res, cb_ref, idx_ref):
    nsteps = K // KB
    r2 = -2.0 * res
    rn = jnp.transpose(jnp.sum(res * res, axis=1, keepdims=True))
    ii = lax.broadcasted_iota(jnp.int32, (KB, NB), 0).astype(jnp.float32)

    def step(k, carry):
        bmin, barg = carry
        cbk = cb_ref[0, pl.ds(k * KB, KB), :]
        m2 = lax.dot_general(cbk, r2, (((1,), (1,)), ((), ())),
                             preferred_element_type=jnp.float32)
        cn = jnp.sum(cbk * cbk, axis=1, keepdims=True)
        s = (rn + m2) + cn
        lmin = jnp.min(s, axis=0, keepdims=True)
        cand = jnp.where(s == lmin, ii, jnp.float32(KB))
        larg_f = jnp.min(cand, axis=0, keepdims=True)
        larg = larg_f.astype(jnp.int32) + k * KB
        improved = lmin < bmin
        return (jnp.where(improved, lmin, bmin),
                jnp.where(improved, larg, barg))

    init = (jnp.full((1, NB), jnp.inf, jnp.float32),
            jnp.zeros((1, NB), jnp.int32))
    carry = init
    for k in range(nsteps):
        carry = step(k, carry)
    idx_ref[0] = carry[1]


def _vq0_body(rep_ref, cb_ref, idx_ref):
    _argmin_over_k(rep_ref[...], cb_ref, idx_ref)


def _vq1_body(rep_ref, q0_ref, cb_ref, idx_ref):
    _argmin_over_k(rep_ref[...] - q0_ref[...], cb_ref, idx_ref)


def _dec_body(x_ref, rep_ref, q0_ref, q1_ref,
              w1_ref, b1_ref, w2_ref, b2_ref, w3_ref, b3_ref, loss_ref):
    q0 = q0_ref[...]
    q1 = q1_ref[...]
    rep = rep_ref[...]
    sv = q0 + q1
    h = jnp.maximum(
        jnp.dot(sv, w1_ref[...], preferred_element_type=jnp.float32) + b1_ref[...], 0.0)
    h = jnp.maximum(
        jnp.dot(h, w2_ref[...], preferred_element_type=jnp.float32) + b2_ref[...], 0.0)
    out = jnp.dot(h, w3_ref[...], preferred_element_type=jnp.float32) + b3_ref[...]

    def _sum2d(a):
        return jnp.sum(jnp.sum(a, axis=1, keepdims=True), axis=0, keepdims=True)

    x = x_ref[...]
    sabs = _sum2d(jnp.abs(x - out))
    e0 = q0 - rep
    e1 = q1 - (rep - q0)
    sse = _sum2d(e0 * e0) + _sum2d(e1 * e1)
    contrib = sabs / (N * TA) + 5.0 * sse / (N * D)

    @pl.when(pl.program_id(0) == 0)
    def _init():
        loss_ref[...] = jnp.zeros_like(loss_ref)

    loss_ref[...] += contrib


def _encode(x, w1, b1, w2, b2, w3, b3):
    full = lambda shape: pl.BlockSpec(shape, lambda n: tuple(0 for _ in shape))
    return pl.pallas_call(
        _enc_body,
        grid=(N // MB,),
        in_specs=[
            pl.BlockSpec((MB, TA), lambda n: (n, 0)),
            full((TA, H)), full((1, H)),
            full((H, H)), full((1, H)),
            full((H, D)), full((1, D)),
        ],
        out_specs=pl.BlockSpec((MB, D), lambda n: (n, 0)),
        out_shape=jax.ShapeDtypeStruct((N, D), jnp.float32),
    )(x, w1, b1, w2, b2, w3, b3)


def _vq_argmin(rep, codebooks, q, q0=None):
    cb_spec = pl.BlockSpec((1, K, D), lambda n: (q, 0, 0))
    row_spec = pl.BlockSpec((NB, D), lambda n: (n, 0))
    idx_spec = pl.BlockSpec((1, 1, NB), lambda n: (n, 0, 0))
    out_shape = jax.ShapeDtypeStruct((GRID, 1, NB), jnp.int32)
    if q0 is None:
        idx3 = pl.pallas_call(
            _vq0_body, grid=(GRID,),
            in_specs=[row_spec, cb_spec],
            out_specs=idx_spec, out_shape=out_shape,
        )(rep, codebooks)
    else:
        idx3 = pl.pallas_call(
            _vq1_body, grid=(GRID,),
            in_specs=[row_spec, row_spec, cb_spec],
            out_specs=idx_spec, out_shape=out_shape,
        )(rep, q0, codebooks)
    return idx3.reshape(N)


def _sc_gather(table, idx):
    info = plsc.get_sparse_core_info()
    nw = info.num_cores * info.num_subcores
    b_per_w = N // nw
    mesh = plsc.VectorSubcoreMesh(core_axis_name="c", subcore_axis_name="s")

    @functools.partial(
        pl.kernel, mesh=mesh,
        out_type=jax.ShapeDtypeStruct((N, D), jnp.float32),
        scratch_types=[
            pltpu.VMEM((b_per_w,), jnp.int32),
            pltpu.VMEM((b_per_w, D), jnp.float32),
            pltpu.SemaphoreType.DMA,
        ],
    )
    def gather_k(table_hbm, idx_hbm, out_hbm, idx_v, rows_v, sem):
        wid = lax.axis_index("s") * info.num_cores + lax.axis_index("c")
        base = wid * b_per_w
        pltpu.sync_copy(idx_hbm.at[pl.ds(base, b_per_w)], idx_v)
        pltpu.async_copy(table_hbm.at[idx_v], rows_v, sem).wait()
        pltpu.sync_copy(rows_v, out_hbm.at[pl.ds(base, b_per_w)])

    return gather_k(table, idx)


def _decode_loss(x, rep, q0, q1, w1, b1, w2, b2, w3, b3):
    full = lambda shape: pl.BlockSpec(shape, lambda n: tuple(0 for _ in shape))
    row = lambda w: pl.BlockSpec((MB, w), lambda n: (n, 0))
    loss = pl.pallas_call(
        _dec_body,
        grid=(N // MB,),
        in_specs=[
            pl.BlockSpec((MB, TA), lambda n: (n, 0)),
            row(D), row(D), row(D),
            full((D, H)), full((1, H)),
            full((H, H)), full((1, H)),
            full((H, TA)), full((1, TA)),
        ],
        out_specs=pl.BlockSpec((1, 1), lambda n: (0, 0)),
        out_shape=jax.ShapeDtypeStruct((1, 1), jnp.float32),
    )(x, rep, q0, q1, w1, b1, w2, b2, w3, b3)
    return loss[0, 0]


def kernel(state, enc_w1, enc_b1, enc_w2, enc_b2, enc_w3, enc_b3,
           dec_w1, dec_b1, dec_w2, dec_b2, dec_w3, dec_b3, codebooks):
    x = state.reshape(N, TA)
    cb2d = codebooks.reshape(2 * K, D)
    rep = _encode(x, enc_w1, enc_b1.reshape(1, H), enc_w2,
                  enc_b2.reshape(1, H), enc_w3, enc_b3.reshape(1, D))
    idx0 = _vq_argmin(rep, codebooks, 0)
    quant0 = _sc_gather(cb2d, idx0)
    idx1 = _vq_argmin(rep, codebooks, 1, q0=quant0)
    quant1 = _sc_gather(cb2d, idx1 + K)
    rep_loss = _decode_loss(x, rep, quant0, quant1,
                            dec_w1, dec_b1.reshape(1, H),
                            dec_w2, dec_b2.reshape(1, H),
                            dec_w3, dec_b3.reshape(1, TA))
    vq_code = jnp.stack([idx0, idx1], axis=-1)
    return rep_loss, vq_code
